# baseline (speedup 1.0000x reference)
import functools

import jax
import jax.numpy as jnp
from jax.experimental import pallas as pl

NUM_CODEBOOKS = 4
NUM_CLUSTERS = 1024
DIM = 64
LANES = 128
ROW_BLOCK = 8192


def _rq_body(data_ref, cb_ref, ids_ref, q_ref):
    data = data_ref[...]
    res = data
    r = data.shape[0]
    col_iota = jax.lax.broadcasted_iota(
        jnp.int32, (r, NUM_CLUSTERS), 1).astype(jnp.float32)
    ids_cols = []
    for l in range(NUM_CODEBOOKS):
        cb = cb_ref[l]
        cbn = jnp.sum(cb * cb, axis=-1)[None, :]
        cbm2 = -2.0 * cb
        cb_hi = cb.astype(jnp.bfloat16)
        r1 = cb - cb_hi.astype(jnp.float32)
        cb_mid = r1.astype(jnp.bfloat16)
        cb_lo = (r1 - cb_mid.astype(jnp.float32)).astype(jnp.bfloat16)
        cb3 = jnp.concatenate([cb_hi, cb_mid, cb_lo], axis=-1)

        dn = jnp.sum(res * res, axis=-1, keepdims=True)
        pm2 = jax.lax.dot_general(
            res, cbm2, (((1,), (1,)), ((), ())),
            preferred_element_type=jnp.float32)
        m = (dn + cbn[:, :LANES]) + pm2[:, :LANES]
        c = col_iota[:, :LANES]
        for j in range(1, NUM_CLUSTERS // LANES):
            sl = slice(j * LANES, (j + 1) * LANES)
            d_j = (dn + cbn[:, sl]) + pm2[:, sl]
            c_j = col_iota[:, sl]
            lt = d_j < m
            m = jnp.where(lt, d_j, m)
            c = jnp.where(lt, c_j, c)
        gmin = jnp.min(m, axis=-1, keepdims=True)
        idx = jnp.min(jnp.where(m == gmin, c, float(NUM_CLUSTERS)),
                      axis=-1, keepdims=True)
        onehot = (col_iota == idx).astype(jnp.bfloat16)
        q3 = jax.lax.dot_general(
            onehot, cb3, (((1,), (0,)), ((), ())),
            preferred_element_type=jnp.float32)
        q = (q3[:, :DIM] + q3[:, DIM:2 * DIM]) + q3[:, 2 * DIM:]
        res = res - q
        ids_cols.append(idx.astype(jnp.int32))
    ids_ref[...] = jnp.concatenate(ids_cols, axis=1)
    q_ref[...] = data - res


@functools.partial(jax.jit, static_argnames=("interpret",))
def kernel(data, codebooks, interpret=False):
    n = data.shape[0]
    grid = (n // ROW_BLOCK,)
    ids, quantized = pl.pallas_call(
        _rq_body,
        grid=grid,
        in_specs=[
            pl.BlockSpec((ROW_BLOCK, DIM), lambda i: (i, 0)),
            pl.BlockSpec((NUM_CODEBOOKS, NUM_CLUSTERS, DIM),
                         lambda i: (0, 0, 0)),
        ],
        out_specs=[
            pl.BlockSpec((ROW_BLOCK, NUM_CODEBOOKS), lambda i: (i, 0)),
            pl.BlockSpec((ROW_BLOCK, DIM), lambda i: (i, 0)),
        ],
        out_shape=[
            jax.ShapeDtypeStruct((n, NUM_CODEBOOKS), jnp.int32),
            jax.ShapeDtypeStruct((n, DIM), jnp.float32),
        ],
        interpret=interpret,
    )(data, codebooks)
    return ids, quantized


# speedup vs baseline: 1.2375x; 1.2375x over previous
import functools

import jax
import jax.numpy as jnp
from jax.experimental import pallas as pl
from jax.experimental.pallas import tpu as pltpu

NUM_CODEBOOKS = 4
NUM_CLUSTERS = 1024
DIM = 64
LANES = 128
ROW_BLOCK = 4096


def _rq_body(data_ref, cb_ref, ids_ref, q_ref, cbn_ref, cbm2_ref, cb3_ref):
    @pl.when(pl.program_id(0) == 0)
    def _prep():
        cb_all = cb_ref[...]
        cbn_ref[...] = jnp.sum(cb_all * cb_all, axis=-1)
        cbm2_ref[...] = -2.0 * cb_all
        cb_hi = cb_all.astype(jnp.bfloat16)
        r1 = cb_all - cb_hi.astype(jnp.float32)
        cb_mid = r1.astype(jnp.bfloat16)
        cb_lo = (r1 - cb_mid.astype(jnp.float32)).astype(jnp.bfloat16)
        cb3_ref[...] = jnp.concatenate([cb_hi, cb_mid, cb_lo], axis=-1)

    data = data_ref[...]
    res = data
    r = data.shape[0]
    col_iota = jax.lax.broadcasted_iota(
        jnp.int32, (r, NUM_CLUSTERS), 1).astype(jnp.float32)
    ids_cols = []
    for l in range(NUM_CODEBOOKS):
        cbn = cbn_ref[l][None, :]
        dn = jnp.sum(res * res, axis=-1, keepdims=True)
        pm2 = jax.lax.dot_general(
            res, cbm2_ref[l], (((1,), (1,)), ((), ())),
            preferred_element_type=jnp.float32)
        m = (dn + cbn[:, :LANES]) + pm2[:, :LANES]
        c = col_iota[:, :LANES]
        for j in range(1, NUM_CLUSTERS // LANES):
            sl = slice(j * LANES, (j + 1) * LANES)
            d_j = (dn + cbn[:, sl]) + pm2[:, sl]
            c_j = col_iota[:, sl]
            lt = d_j < m
            m = jnp.where(lt, d_j, m)
            c = jnp.where(lt, c_j, c)
        gmin = jnp.min(m, axis=-1, keepdims=True)
        idx = jnp.min(jnp.where(m == gmin, c, float(NUM_CLUSTERS)),
                      axis=-1, keepdims=True)
        onehot = (col_iota == idx).astype(jnp.bfloat16)
        q3 = jax.lax.dot_general(
            onehot, cb3_ref[l], (((1,), (0,)), ((), ())),
            preferred_element_type=jnp.float32)
        q = (q3[:, :DIM] + q3[:, DIM:2 * DIM]) + q3[:, 2 * DIM:]
        res = res - q
        ids_cols.append(idx.astype(jnp.int32))
    ids_ref[...] = jnp.concatenate(ids_cols, axis=1)
    q_ref[...] = data - res


@functools.partial(jax.jit, static_argnames=("interpret",))
def kernel(data, codebooks, interpret=False):
    n = data.shape[0]
    grid = (n // ROW_BLOCK,)
    ids, quantized = pl.pallas_call(
        _rq_body,
        grid=grid,
        in_specs=[
            pl.BlockSpec((ROW_BLOCK, DIM), lambda i: (i, 0)),
            pl.BlockSpec((NUM_CODEBOOKS, NUM_CLUSTERS, DIM),
                         lambda i: (0, 0, 0)),
        ],
        out_specs=[
            pl.BlockSpec((ROW_BLOCK, NUM_CODEBOOKS), lambda i: (i, 0)),
            pl.BlockSpec((ROW_BLOCK, DIM), lambda i: (i, 0)),
        ],
        out_shape=[
            jax.ShapeDtypeStruct((n, NUM_CODEBOOKS), jnp.int32),
            jax.ShapeDtypeStruct((n, DIM), jnp.float32),
        ],
        scratch_shapes=[
            pltpu.VMEM((NUM_CODEBOOKS, NUM_CLUSTERS), jnp.float32),
            pltpu.VMEM((NUM_CODEBOOKS, NUM_CLUSTERS, DIM), jnp.float32),
            pltpu.VMEM((NUM_CODEBOOKS, NUM_CLUSTERS, 3 * DIM), jnp.bfloat16),
        ],
        interpret=interpret,
    )(data, codebooks)
    return ids, quantized


# speedup vs baseline: 1.2555x; 1.0145x over previous
import functools

import jax
import jax.numpy as jnp
from jax.experimental import pallas as pl
from jax.experimental.pallas import tpu as pltpu

NUM_CODEBOOKS = 4
NUM_CLUSTERS = 1024
DIM = 64
LANES = 128
ROW_BLOCK = 4096


def _rq_body(data_ref, cb_ref, ids_ref, q_ref, cbn_ref, cbm2_ref, cb3_ref):
    @pl.when(pl.program_id(0) == 0)
    def _prep():
        cb_all = cb_ref[...]
        cbn_ref[...] = jnp.sum(cb_all * cb_all, axis=-1)
        cbm2_ref[...] = -2.0 * cb_all
        cb_hi = cb_all.astype(jnp.bfloat16)
        r1 = cb_all - cb_hi.astype(jnp.float32)
        cb_mid = r1.astype(jnp.bfloat16)
        cb_lo = (r1 - cb_mid.astype(jnp.float32)).astype(jnp.bfloat16)
        cb3_ref[...] = jnp.concatenate([cb_hi, cb_mid, cb_lo], axis=-1)

    data = data_ref[...]
    res = data
    r = data.shape[0]
    col_iota = jax.lax.broadcasted_iota(
        jnp.int32, (r, NUM_CLUSTERS), 1).astype(jnp.float32)
    ids_cols = []
    for l in range(NUM_CODEBOOKS):
        cbn = cbn_ref[l][None, :]
        dn = jnp.sum(res * res, axis=-1, keepdims=True)
        pm2 = jax.lax.dot_general(
            res, cbm2_ref[l], (((1,), (1,)), ((), ())),
            preferred_element_type=jnp.float32)
        m = (dn + cbn[:, :LANES]) + pm2[:, :LANES]
        c = col_iota[:, :LANES]
        for j in range(1, NUM_CLUSTERS // LANES):
            sl = slice(j * LANES, (j + 1) * LANES)
            d_j = (dn + cbn[:, sl]) + pm2[:, sl]
            c_j = col_iota[:, sl]
            lt = d_j < m
            m = jnp.minimum(m, d_j)
            c = jnp.where(lt, c_j, c)
        gmin = jnp.min(m, axis=-1, keepdims=True)
        idx = jnp.min(jnp.where(m == gmin, c, float(NUM_CLUSTERS)),
                      axis=-1, keepdims=True)
        onehot = (col_iota == idx).astype(jnp.bfloat16)
        q3 = jax.lax.dot_general(
            onehot, cb3_ref[l], (((1,), (0,)), ((), ())),
            preferred_element_type=jnp.float32)
        q = (q3[:, :DIM] + q3[:, DIM:2 * DIM]) + q3[:, 2 * DIM:]
        res = res - q
        ids_cols.append(idx.astype(jnp.int32))
    ids_ref[...] = jnp.concatenate(ids_cols, axis=1)
    q_ref[...] = data - res


@functools.partial(jax.jit, static_argnames=("interpret",))
def kernel(data, codebooks, interpret=False):
    n = data.shape[0]
    grid = (n // ROW_BLOCK,)
    ids, quantized = pl.pallas_call(
        _rq_body,
        grid=grid,
        in_specs=[
            pl.BlockSpec((ROW_BLOCK, DIM), lambda i: (i, 0)),
            pl.BlockSpec((NUM_CODEBOOKS, NUM_CLUSTERS, DIM),
                         lambda i: (0, 0, 0)),
        ],
        out_specs=[
            pl.BlockSpec((ROW_BLOCK, NUM_CODEBOOKS), lambda i: (i, 0)),
            pl.BlockSpec((ROW_BLOCK, DIM), lambda i: (i, 0)),
        ],
        out_shape=[
            jax.ShapeDtypeStruct((n, NUM_CODEBOOKS), jnp.int32),
            jax.ShapeDtypeStruct((n, DIM), jnp.float32),
        ],
        scratch_shapes=[
            pltpu.VMEM((NUM_CODEBOOKS, NUM_CLUSTERS), jnp.float32),
            pltpu.VMEM((NUM_CODEBOOKS, NUM_CLUSTERS, DIM), jnp.float32),
            pltpu.VMEM((NUM_CODEBOOKS, NUM_CLUSTERS, 3 * DIM), jnp.bfloat16),
        ],
        interpret=interpret,
    )(data, codebooks)
    return ids, quantized


# speedup vs baseline: 1.2686x; 1.0105x over previous
import functools

import jax
import jax.numpy as jnp
from jax.experimental import pallas as pl
from jax.experimental.pallas import tpu as pltpu

NUM_CODEBOOKS = 4
NUM_CLUSTERS = 1024
DIM = 64
LANES = 128
ROW_BLOCK = 4096


def _rq_body(data_ref, cb_ref, ids_ref, q_ref, cbn_ref, cbm2_ref, cb3_ref):
    @pl.when(pl.program_id(0) == 0)
    def _prep():
        cb_all = cb_ref[...]
        cbn_ref[...] = jnp.sum(cb_all * cb_all, axis=-1)
        cbm2_ref[...] = -2.0 * cb_all
        cb_hi = cb_all.astype(jnp.bfloat16)
        r1 = cb_all - cb_hi.astype(jnp.float32)
        cb_mid = r1.astype(jnp.bfloat16)
        cb_lo = (r1 - cb_mid.astype(jnp.float32)).astype(jnp.bfloat16)
        cb3_ref[...] = jnp.concatenate([cb_hi, cb_mid, cb_lo], axis=-1)

    data = data_ref[...]
    res = data
    r = data.shape[0]
    col_iota = jax.lax.broadcasted_iota(
        jnp.int32, (r, NUM_CLUSTERS), 1).astype(jnp.float32)
    lane_iota = jax.lax.broadcasted_iota(
        jnp.int32, (r, LANES), 1).astype(jnp.float32)
    ids_cols = []
    for l in range(NUM_CODEBOOKS):
        cbn = cbn_ref[l][None, :]
        dn = jnp.sum(res * res, axis=-1, keepdims=True)
        pm2 = jax.lax.dot_general(
            res, cbm2_ref[l], (((1,), (1,)), ((), ())),
            preferred_element_type=jnp.float32)
        m = (dn + cbn[:, :LANES]) + pm2[:, :LANES]
        bj = jnp.zeros((r, LANES), jnp.float32)
        for j in range(1, NUM_CLUSTERS // LANES):
            sl = slice(j * LANES, (j + 1) * LANES)
            d_j = (dn + cbn[:, sl]) + pm2[:, sl]
            lt = d_j < m
            m = jnp.minimum(m, d_j)
            bj = jnp.where(lt, float(j), bj)
        gmin = jnp.min(m, axis=-1, keepdims=True)
        code = bj * float(LANES) + lane_iota
        idx = jnp.min(jnp.where(m == gmin, code, float(NUM_CLUSTERS)),
                      axis=-1, keepdims=True)
        onehot = (col_iota == idx).astype(jnp.bfloat16)
        q3 = jax.lax.dot_general(
            onehot, cb3_ref[l], (((1,), (0,)), ((), ())),
            preferred_element_type=jnp.float32)
        q = (q3[:, :DIM] + q3[:, DIM:2 * DIM]) + q3[:, 2 * DIM:]
        res = res - q
        ids_cols.append(idx.astype(jnp.int32))
    ids_ref[...] = jnp.concatenate(ids_cols, axis=1)
    q_ref[...] = data - res


@functools.partial(jax.jit, static_argnames=("interpret",))
def kernel(data, codebooks, interpret=False):
    n = data.shape[0]
    grid = (n // ROW_BLOCK,)
    ids, quantized = pl.pallas_call(
        _rq_body,
        grid=grid,
        in_specs=[
            pl.BlockSpec((ROW_BLOCK, DIM), lambda i: (i, 0)),
            pl.BlockSpec((NUM_CODEBOOKS, NUM_CLUSTERS, DIM),
                         lambda i: (0, 0, 0)),
        ],
        out_specs=[
            pl.BlockSpec((ROW_BLOCK, NUM_CODEBOOKS), lambda i: (i, 0)),
            pl.BlockSpec((ROW_BLOCK, DIM), lambda i: (i, 0)),
        ],
        out_shape=[
            jax.ShapeDtypeStruct((n, NUM_CODEBOOKS), jnp.int32),
            jax.ShapeDtypeStruct((n, DIM), jnp.float32),
        ],
        scratch_shapes=[
            pltpu.VMEM((NUM_CODEBOOKS, NUM_CLUSTERS), jnp.float32),
            pltpu.VMEM((NUM_CODEBOOKS, NUM_CLUSTERS, DIM), jnp.float32),
            pltpu.VMEM((NUM_CODEBOOKS, NUM_CLUSTERS, 3 * DIM), jnp.bfloat16),
        ],
        interpret=interpret,
    )(data, codebooks)
    return ids, quantized
